# baseline (speedup 1.0000x reference)
import functools

import jax
import jax.numpy as jnp
from jax import lax
from jax.experimental import pallas as pl
from jax.experimental.pallas import tpu as pltpu
from jax.experimental.pallas import tpu_sc as plsc

N = 10000
E = 160000
IN_DIM = 128
NUM_ETYPES = 4
OUT = 32

NC = 2
NS = 16
NW = NC * NS
CHUNK = 128
CHUNKS_PER_W = 40
E_PAD = NW * CHUNKS_PER_W * CHUNK
N_PAD = 10240
ROWS_PER_S = N_PAD // NS
ACC_ROWS = N_PAD


def _mm_body(x_ref, wcat_ref, rt_ref, b_ref, y_ref, res_ref):
    xb = x_ref[...]
    y_ref[...] = jnp.dot(xb, wcat_ref[...], preferred_element_type=jnp.float32)
    res_ref[...] = (
        jnp.dot(xb, rt_ref[...], preferred_element_type=jnp.float32) + b_ref[...]
    )


def _typed_mm(x, wcat, rt, bias2d):
    bn = 2000
    grid = N // bn
    return pl.pallas_call(
        _mm_body,
        grid=(grid,),
        in_specs=[
            pl.BlockSpec((bn, IN_DIM), lambda i: (i, 0)),
            pl.BlockSpec((IN_DIM, NUM_ETYPES * OUT), lambda i: (0, 0)),
            pl.BlockSpec((IN_DIM, OUT), lambda i: (0, 0)),
            pl.BlockSpec((1, OUT), lambda i: (0, 0)),
        ],
        out_specs=[
            pl.BlockSpec((bn, NUM_ETYPES * OUT), lambda i: (i, 0)),
            pl.BlockSpec((bn, OUT), lambda i: (i, 0)),
        ],
        out_shape=[
            jax.ShapeDtypeStruct((N, NUM_ETYPES * OUT), jnp.float32),
            jax.ShapeDtypeStruct((N, OUT), jnp.float32),
        ],
    )(x, wcat, rt, bias2d)


NBUF = 4
K0 = 56
K1 = 80 - K0
KMAX = max(K0, K1)
TOTAL_CHUNKS = E_PAD // CHUNK


def _sc_body(ytab, gidx_h, dst_h, out, gidx_v, dst_v, rows_v, drain_v, acc,
             gsem):
    c = lax.axis_index("c")
    s = lax.axis_index("s")

    zeros16 = jnp.zeros((16,), jnp.float32)

    def _zero(r, _):
        rows_v[0, r, pl.ds(0, 16)] = zeros16
        rows_v[0, r, pl.ds(16, 16)] = zeros16
        return 0

    with jax.named_scope("ph_init"):
        lax.fori_loop(0, CHUNK, _zero, 0)
        for k in range(ROWS_PER_S // CHUNK):
            pltpu.sync_copy(
                rows_v.at[0], acc.at[pl.ds(s * ROWS_PER_S + k * CHUNK, CHUNK)])
        plsc.subcore_barrier()

    def _work(base, nch):
        with jax.named_scope("ph_idx"):
            pltpu.sync_copy(gidx_h.at[pl.ds(base, nch)],
                            gidx_v.at[pl.ds(0, nch)])
            pltpu.sync_copy(dst_h.at[pl.ds(base, nch)],
                            dst_v.at[pl.ds(0, nch)])

        def _round(g, _):
            descs = []
            for b in range(NBUF):
                descs.append(pltpu.async_copy(
                    ytab.at[gidx_v.at[g * NBUF + b]], rows_v.at[b],
                    gsem.at[b]))
            for b in range(NBUF):
                descs[b].wait()
                pltpu.sync_copy(
                    rows_v.at[b], acc.at[dst_v.at[g * NBUF + b]], add=True)
            return 0

        with jax.named_scope("ph_loop"):
            lax.fori_loop(0, nch // NBUF, _round, 0, unroll=False)

    @pl.when(c == 0)
    def _():
        _work(s * K0, K0)

    @pl.when(c == 1)
    def _():
        _work(NS * K0 + s * K1, K1)

    with jax.named_scope("ph_bar2"):
        plsc.subcore_barrier()

    with jax.named_scope("ph_drain"):
        pltpu.sync_copy(acc.at[pl.ds(s * ROWS_PER_S, ROWS_PER_S)], drain_v)
        pltpu.sync_copy(drain_v, out.at[c, pl.ds(s * ROWS_PER_S, ROWS_PER_S)])


@functools.cache
def _scatter_kernel():
  return pl.kernel(
    _sc_body,
    out_type=jax.ShapeDtypeStruct((NC, N_PAD, OUT), jnp.float32),
    mesh=plsc.VectorSubcoreMesh(
        core_axis_name="c", subcore_axis_name="s",
        num_cores=NC, num_subcores=NS),
    scratch_types=[
        pltpu.VMEM((KMAX, CHUNK), jnp.int32),
        pltpu.VMEM((KMAX, CHUNK), jnp.int32),
        pltpu.VMEM((NBUF, CHUNK, OUT), jnp.float32),
        pltpu.VMEM((ROWS_PER_S, OUT), jnp.float32),
        pltpu.VMEM_SHARED((ACC_ROWS, OUT), jnp.float32),
        pltpu.SemaphoreType.DMA((NBUF,)),
    ],
    compiler_params=pltpu.CompilerParams(use_tc_tiling_on_sc=False),
  )


def _combine_body(p_ref, res_ref, o_ref):
    o_ref[...] = p_ref[0] + p_ref[1] + res_ref[...]


def _combine(partials, res):
    bn = 2000
    return pl.pallas_call(
        _combine_body,
        grid=(N // bn,),
        in_specs=[
            pl.BlockSpec((NC, bn, OUT), lambda i: (0, i, 0)),
            pl.BlockSpec((bn, OUT), lambda i: (i, 0)),
        ],
        out_specs=pl.BlockSpec((bn, OUT), lambda i: (i, 0)),
        out_shape=jax.ShapeDtypeStruct((N, OUT), jnp.float32),
    )(partials, res)


def kernel(x, edge_index, ntype, etype, W_weight, al_weight, ar_weight,
           res_weight, res_bias):
    del ntype, al_weight, ar_weight

    src = edge_index[0]
    dst = edge_index[1]

    wcat = jnp.transpose(W_weight, (1, 0, 2)).reshape(IN_DIM, NUM_ETYPES * OUT)
    y, res = _typed_mm(x, wcat, res_weight.T, res_bias.reshape(1, OUT))
    ytab = y.reshape(N * NUM_ETYPES, OUT)

    gidx = src * NUM_ETYPES + etype
    pad = E_PAD - E
    gidx_p = jnp.concatenate([gidx, jnp.zeros((pad,), jnp.int32)])
    junk = N + (jnp.arange(pad, dtype=jnp.int32) % (ACC_ROWS - N))
    dst_p = jnp.concatenate([dst, junk])
    gidx_p = gidx_p.reshape(TOTAL_CHUNKS, CHUNK)
    dst_p = dst_p.reshape(TOTAL_CHUNKS, CHUNK)

    partials = _scatter_kernel()(ytab, gidx_p, dst_p)
    return _combine(partials, res)


# speedup vs baseline: 1.3414x; 1.3414x over previous
import functools

import jax
import jax.numpy as jnp
from jax import lax
from jax.experimental import pallas as pl
from jax.experimental.pallas import tpu as pltpu
from jax.experimental.pallas import tpu_sc as plsc

N = 10000
E = 160000
IN_DIM = 128
NUM_ETYPES = 4
OUT = 32

NC = 2
NS = 16
NW = NC * NS
CHUNK = 128
CHUNKS_PER_W = 40
E_PAD = NW * CHUNKS_PER_W * CHUNK
N_PAD = 10240
ROWS_PER_S = N_PAD // NS
ACC_ROWS = N_PAD


def _mm_body(x_ref, wcat_ref, rt_ref, b_ref, y_ref, res_ref):
    xb = x_ref[...]
    y_ref[...] = jnp.dot(xb, wcat_ref[...], preferred_element_type=jnp.float32)
    res_ref[...] = (
        jnp.dot(xb, rt_ref[...], preferred_element_type=jnp.float32) + b_ref[...]
    )


def _typed_mm(x, wcat, rt, bias2d):
    bn = 2000
    grid = N // bn
    return pl.pallas_call(
        _mm_body,
        grid=(grid,),
        in_specs=[
            pl.BlockSpec((bn, IN_DIM), lambda i: (i, 0)),
            pl.BlockSpec((IN_DIM, NUM_ETYPES * OUT), lambda i: (0, 0)),
            pl.BlockSpec((IN_DIM, OUT), lambda i: (0, 0)),
            pl.BlockSpec((1, OUT), lambda i: (0, 0)),
        ],
        out_specs=[
            pl.BlockSpec((bn, NUM_ETYPES * OUT), lambda i: (i, 0)),
            pl.BlockSpec((bn, OUT), lambda i: (i, 0)),
        ],
        out_shape=[
            jax.ShapeDtypeStruct((N, NUM_ETYPES * OUT), jnp.float32),
            jax.ShapeDtypeStruct((N, OUT), jnp.float32),
        ],
    )(x, wcat, rt, bias2d)


NBUF = 4
K0 = 40
K1 = 80 - K0
KMAX = max(K0, K1)
TOTAL_CHUNKS = E_PAD // CHUNK


def _sc_body(ytab, gidx_h, dst_h, out, gidx_v, dst_v, rows_v, drain_v, acc,
             gsem):
    c = lax.axis_index("c")
    s = lax.axis_index("s")

    zeros16 = jnp.zeros((16,), jnp.float32)

    def _zero(r, _):
        rows_v[0, r, pl.ds(0, 16)] = zeros16
        rows_v[0, r, pl.ds(16, 16)] = zeros16
        return 0

    lax.fori_loop(0, CHUNK, _zero, 0)
    for k in range(ROWS_PER_S // CHUNK):
        pltpu.sync_copy(
            rows_v.at[0], acc.at[pl.ds(s * ROWS_PER_S + k * CHUNK, CHUNK)])
    plsc.subcore_barrier()

    def _work(base, nch):
        pltpu.sync_copy(gidx_h.at[pl.ds(base, nch)], gidx_v.at[pl.ds(0, nch)])
        pltpu.sync_copy(dst_h.at[pl.ds(base, nch)], dst_v.at[pl.ds(0, nch)])

        def _round(g, _):
            descs = []
            for b in range(NBUF):
                descs.append(pltpu.async_copy(
                    ytab.at[gidx_v.at[g * NBUF + b]], rows_v.at[b],
                    gsem.at[b]))
            for b in range(NBUF):
                descs[b].wait()
                pltpu.sync_copy(
                    rows_v.at[b], acc.at[dst_v.at[g * NBUF + b]], add=True)
            return 0

        lax.fori_loop(0, nch // NBUF, _round, 0, unroll=False)

    @pl.when(c == 0)
    def _():
        _work(s * K0, K0)

    @pl.when(c == 1)
    def _():
        _work(NS * K0 + s * K1, K1)

    plsc.subcore_barrier()

    pltpu.sync_copy(acc.at[pl.ds(s * ROWS_PER_S, ROWS_PER_S)], drain_v)
    pltpu.sync_copy(drain_v, out.at[c, pl.ds(s * ROWS_PER_S, ROWS_PER_S)])


@functools.cache
def _scatter_kernel():
  return pl.kernel(
    _sc_body,
    out_type=jax.ShapeDtypeStruct((NC, N_PAD, OUT), jnp.float32),
    mesh=plsc.VectorSubcoreMesh(
        core_axis_name="c", subcore_axis_name="s",
        num_cores=NC, num_subcores=NS),
    scratch_types=[
        pltpu.VMEM((KMAX, CHUNK), jnp.int32),
        pltpu.VMEM((KMAX, CHUNK), jnp.int32),
        pltpu.VMEM((NBUF, CHUNK, OUT), jnp.float32),
        pltpu.VMEM((ROWS_PER_S, OUT), jnp.float32),
        pltpu.VMEM_SHARED((ACC_ROWS, OUT), jnp.float32),
        pltpu.SemaphoreType.DMA((NBUF,)),
    ],
    compiler_params=pltpu.CompilerParams(use_tc_tiling_on_sc=False),
  )


def _combine_body(p_ref, res_ref, o_ref):
    o_ref[...] = p_ref[0] + p_ref[1] + res_ref[...]


def _combine(partials, res):
    bn = 2000
    return pl.pallas_call(
        _combine_body,
        grid=(N // bn,),
        in_specs=[
            pl.BlockSpec((NC, bn, OUT), lambda i: (0, i, 0)),
            pl.BlockSpec((bn, OUT), lambda i: (i, 0)),
        ],
        out_specs=pl.BlockSpec((bn, OUT), lambda i: (i, 0)),
        out_shape=jax.ShapeDtypeStruct((N, OUT), jnp.float32),
    )(partials, res)


def kernel(x, edge_index, ntype, etype, W_weight, al_weight, ar_weight,
           res_weight, res_bias):
    del ntype, al_weight, ar_weight

    src = edge_index[0]
    dst = edge_index[1]

    wcat = jnp.transpose(W_weight, (1, 0, 2)).reshape(IN_DIM, NUM_ETYPES * OUT)
    y, res = _typed_mm(x, wcat, res_weight.T, res_bias.reshape(1, OUT))
    ytab = y.reshape(N * NUM_ETYPES, OUT)

    gidx = src * NUM_ETYPES + etype
    pad = E_PAD - E
    padi = jnp.arange(pad, dtype=jnp.int32)
    gidx_p = jnp.concatenate([gidx, padi % (N * NUM_ETYPES)])
    dst_p = jnp.concatenate([dst, N + padi % (ACC_ROWS - N)])
    gidx_p = gidx_p.reshape(TOTAL_CHUNKS, CHUNK)
    dst_p = dst_p.reshape(TOTAL_CHUNKS, CHUNK)

    partials = _scatter_kernel()(ytab, gidx_p, dst_p)
    return _combine(partials, res)


# speedup vs baseline: 1.5546x; 1.1590x over previous
import functools

import jax
import jax.numpy as jnp
from jax import lax
from jax.experimental import pallas as pl
from jax.experimental.pallas import tpu as pltpu
from jax.experimental.pallas import tpu_sc as plsc

N = 10000
E = 160000
IN_DIM = 128
NUM_ETYPES = 4
OUT = 32

NC = 2
NS = 16
NW = NC * NS
CHUNK = 128
CHUNKS_PER_W = 40
E_PAD = NW * CHUNKS_PER_W * CHUNK
N_PAD = 10240
ROWS_PER_S = N_PAD // NS
ACC_ROWS = N_PAD


def _mm_body(x_ref, wcat_ref, rt_ref, b_ref, y_ref, res_ref):
    xb = x_ref[...]
    y_ref[...] = jnp.dot(xb, wcat_ref[...], preferred_element_type=jnp.float32)
    res_ref[...] = (
        jnp.dot(xb, rt_ref[...], preferred_element_type=jnp.float32) + b_ref[...]
    )


def _typed_mm(x, wcat, rt, bias2d):
    bn = 2000
    grid = N // bn
    return pl.pallas_call(
        _mm_body,
        grid=(grid,),
        in_specs=[
            pl.BlockSpec((bn, IN_DIM), lambda i: (i, 0)),
            pl.BlockSpec((IN_DIM, NUM_ETYPES * OUT), lambda i: (0, 0)),
            pl.BlockSpec((IN_DIM, OUT), lambda i: (0, 0)),
            pl.BlockSpec((1, OUT), lambda i: (0, 0)),
        ],
        out_specs=[
            pl.BlockSpec((bn, NUM_ETYPES * OUT), lambda i: (i, 0)),
            pl.BlockSpec((bn, OUT), lambda i: (i, 0)),
        ],
        out_shape=[
            jax.ShapeDtypeStruct((N, NUM_ETYPES * OUT), jnp.float32),
            jax.ShapeDtypeStruct((N, OUT), jnp.float32),
        ],
    )(x, wcat, rt, bias2d)


NBUF = 4
K0 = 40
K1 = 80 - K0
KMAX = max(K0, K1)
TOTAL_CHUNKS = E_PAD // CHUNK


def _sc_body(ytab, gidx_h, dst_h, out, gidx_v, dst_v, rows_v, drain_v, acc,
             gsem):
    c = lax.axis_index("c")
    s = lax.axis_index("s")

    zeros16 = jnp.zeros((16,), jnp.float32)

    def _zero(r, _):
        rows_v[0, r, pl.ds(0, 16)] = zeros16
        rows_v[0, r, pl.ds(16, 16)] = zeros16
        return 0

    lax.fori_loop(0, CHUNK, _zero, 0)
    for k in range(ROWS_PER_S // CHUNK):
        pltpu.sync_copy(
            rows_v.at[0], acc.at[pl.ds(s * ROWS_PER_S + k * CHUNK, CHUNK)])
    plsc.subcore_barrier()

    def _work(base, nch):
        pltpu.sync_copy(gidx_h.at[pl.ds(base, nch)], gidx_v.at[pl.ds(0, nch)])
        pltpu.sync_copy(dst_h.at[pl.ds(base, nch)], dst_v.at[pl.ds(0, nch)])

        def _round(g, _):
            descs = []
            for b in range(NBUF):
                descs.append(pltpu.async_copy(
                    ytab.at[gidx_v.at[g * NBUF + b]], rows_v.at[b],
                    gsem.at[b]))
            for b in range(NBUF):
                descs[b].wait()
                pltpu.sync_copy(
                    rows_v.at[b], acc.at[dst_v.at[g * NBUF + b]], add=True)
            return 0

        lax.fori_loop(0, nch // NBUF, _round, 0, unroll=False)

    @pl.when(c == 0)
    def _():
        _work(s * K0, K0)

    @pl.when(c == 1)
    def _():
        _work(NS * K0 + s * K1, K1)

    plsc.subcore_barrier()

    pltpu.sync_copy(acc.at[pl.ds(s * ROWS_PER_S, ROWS_PER_S)], drain_v)
    pltpu.sync_copy(drain_v, out.at[c, pl.ds(s * ROWS_PER_S, ROWS_PER_S)])


@functools.cache
def _scatter_kernel():
  return pl.kernel(
    _sc_body,
    out_type=jax.ShapeDtypeStruct((NC, N_PAD, OUT), jnp.float32),
    mesh=plsc.VectorSubcoreMesh(
        core_axis_name="c", subcore_axis_name="s",
        num_cores=NC, num_subcores=NS),
    scratch_types=[
        pltpu.VMEM((KMAX, CHUNK), jnp.int32),
        pltpu.VMEM((KMAX, CHUNK), jnp.int32),
        pltpu.VMEM((NBUF, CHUNK, OUT), jnp.float32),
        pltpu.VMEM((ROWS_PER_S, OUT), jnp.float32),
        pltpu.VMEM_SHARED((ACC_ROWS, OUT), jnp.float32),
        pltpu.SemaphoreType.DMA((NBUF,)),
    ],
    compiler_params=pltpu.CompilerParams(use_tc_tiling_on_sc=False),
  )


NW128 = N * OUT // 128


def _combine_body(p_ref, res_ref, o_ref):
    o_ref[...] = p_ref[0, :NW128] + p_ref[1, :NW128] + res_ref[...]


def _combine(partials, res128):
    p128 = partials.reshape(NC, N_PAD * OUT // 128, 128)
    return pl.pallas_call(
        _combine_body,
        out_shape=jax.ShapeDtypeStruct((NW128, 128), jnp.float32),
    )(p128, res128)


def kernel(x, edge_index, ntype, etype, W_weight, al_weight, ar_weight,
           res_weight, res_bias):
    del ntype, al_weight, ar_weight

    src = edge_index[0]
    dst = edge_index[1]

    wcat = jnp.transpose(W_weight, (1, 0, 2)).reshape(IN_DIM, NUM_ETYPES * OUT)
    y, res = _typed_mm(x, wcat, res_weight.T, res_bias.reshape(1, OUT))
    ytab = y.reshape(N * NUM_ETYPES, OUT)

    gidx = src * NUM_ETYPES + etype
    pad = E_PAD - E
    padi = jnp.arange(pad, dtype=jnp.int32)
    gidx_p = jnp.concatenate([gidx, padi % (N * NUM_ETYPES)])
    dst_p = jnp.concatenate([dst, N + padi % (ACC_ROWS - N)])
    gidx_p = gidx_p.reshape(TOTAL_CHUNKS, CHUNK)
    dst_p = dst_p.reshape(TOTAL_CHUNKS, CHUNK)

    partials = _scatter_kernel()(ytab, gidx_p, dst_p)
    out128 = _combine(partials, res.reshape(NW128, 128))
    return out128.reshape(N, OUT)


# speedup vs baseline: 1.6792x; 1.0801x over previous
import functools

import jax
import jax.numpy as jnp
from jax import lax
from jax.experimental import pallas as pl
from jax.experimental.pallas import tpu as pltpu
from jax.experimental.pallas import tpu_sc as plsc

N = 10000
E = 160000
IN_DIM = 128
NUM_ETYPES = 4
OUT = 32

NC = 2
NS = 16
NW = NC * NS
CHUNK = 128
N_PAD = 10240
ROWS_PER_S = N_PAD // NS
ACC_ROWS = N_PAD


def _mm_body(x_ref, wcat_ref, rt_ref, b_ref, y_ref, res_ref):
    xb = x_ref[...]
    y_ref[...] = jnp.dot(xb, wcat_ref[...], preferred_element_type=jnp.float32)
    res_ref[...] = (
        jnp.dot(xb, rt_ref[...], preferred_element_type=jnp.float32) + b_ref[...]
    )


def _typed_mm(x, wcat, rt, bias2d):
    bn = 2000
    grid = N // bn
    return pl.pallas_call(
        _mm_body,
        grid=(grid,),
        in_specs=[
            pl.BlockSpec((bn, IN_DIM), lambda i: (i, 0)),
            pl.BlockSpec((IN_DIM, NUM_ETYPES * OUT), lambda i: (0, 0)),
            pl.BlockSpec((IN_DIM, OUT), lambda i: (0, 0)),
            pl.BlockSpec((1, OUT), lambda i: (0, 0)),
        ],
        out_specs=[
            pl.BlockSpec((bn, NUM_ETYPES * OUT), lambda i: (i, 0)),
            pl.BlockSpec((bn, OUT), lambda i: (i, 0)),
        ],
        out_shape=[
            jax.ShapeDtypeStruct((N, NUM_ETYPES * OUT), jnp.float32),
            jax.ShapeDtypeStruct((N, OUT), jnp.float32),
        ],
    )(x, wcat, rt, bias2d)


NBUF = 4
KW = 40
EPW = KW * CHUNK
KLAST = (E - (NW - 1) * EPW) // CHUNK


def _sc_body(ei, et, ytab, out, src_st, et_st, dst_st, gidx_v, dst_v, rows_v,
             drain_v, acc, gsem):
    c = lax.axis_index("c")
    s = lax.axis_index("s")
    wid = c * NS + s

    zeros16 = jnp.zeros((16,), jnp.float32)

    def _zero(r, _):
        rows_v[0, r, pl.ds(0, 16)] = zeros16
        rows_v[0, r, pl.ds(16, 16)] = zeros16
        return 0

    lax.fori_loop(0, CHUNK, _zero, 0)
    for k in range(ROWS_PER_S // CHUNK):
        pltpu.sync_copy(
            rows_v.at[0], acc.at[pl.ds(s * ROWS_PER_S + k * CHUNK, CHUNK)])
    plsc.subcore_barrier()

    def _work(base, nch):
        ne = nch * CHUNK
        pltpu.sync_copy(ei.at[0, pl.ds(base, ne)], src_st.at[pl.ds(0, ne)])
        pltpu.sync_copy(ei.at[1, pl.ds(base, ne)], dst_st.at[pl.ds(0, ne)])
        pltpu.sync_copy(et.at[pl.ds(base, ne)], et_st.at[pl.ds(0, ne)])

        def _mkidx(j, _):
            for b in range(CHUNK // 16):
                o = j * CHUNK + b * 16
                gidx_v[j, pl.ds(b * 16, 16)] = (
                    src_st[pl.ds(o, 16)] * NUM_ETYPES + et_st[pl.ds(o, 16)])
                dst_v[j, pl.ds(b * 16, 16)] = dst_st[pl.ds(o, 16)]
            return 0

        lax.fori_loop(0, nch, _mkidx, 0)

        def _round(g, _):
            descs = []
            for b in range(NBUF):
                descs.append(pltpu.async_copy(
                    ytab.at[gidx_v.at[g * NBUF + b]], rows_v.at[b],
                    gsem.at[b]))
            for b in range(NBUF):
                descs[b].wait()
                pltpu.sync_copy(
                    rows_v.at[b], acc.at[dst_v.at[g * NBUF + b]], add=True)
            return 0

        lax.fori_loop(0, nch // NBUF, _round, 0, unroll=False)
        for b in range(nch % NBUF):
            d = pltpu.async_copy(
                ytab.at[gidx_v.at[(nch // NBUF) * NBUF + b]], rows_v.at[b],
                gsem.at[b])
            d.wait()
            pltpu.sync_copy(
                rows_v.at[b], acc.at[dst_v.at[(nch // NBUF) * NBUF + b]],
                add=True)

    @pl.when(wid < NW - 1)
    def _():
        _work(wid * EPW, KW)

    @pl.when(wid == NW - 1)
    def _():
        _work((NW - 1) * EPW, KLAST)

    plsc.subcore_barrier()

    pltpu.sync_copy(acc.at[pl.ds(s * ROWS_PER_S, ROWS_PER_S)], drain_v)
    pltpu.sync_copy(drain_v, out.at[c, pl.ds(s * ROWS_PER_S, ROWS_PER_S)])


@functools.cache
def _scatter_kernel():
  return pl.kernel(
    _sc_body,
    out_type=jax.ShapeDtypeStruct((NC, N_PAD, OUT), jnp.float32),
    mesh=plsc.VectorSubcoreMesh(
        core_axis_name="c", subcore_axis_name="s",
        num_cores=NC, num_subcores=NS),
    scratch_types=[
        pltpu.VMEM((EPW,), jnp.int32),
        pltpu.VMEM((EPW,), jnp.int32),
        pltpu.VMEM((EPW,), jnp.int32),
        pltpu.VMEM((KW, CHUNK), jnp.int32),
        pltpu.VMEM((KW, CHUNK), jnp.int32),
        pltpu.VMEM((NBUF, CHUNK, OUT), jnp.float32),
        pltpu.VMEM((ROWS_PER_S, OUT), jnp.float32),
        pltpu.VMEM_SHARED((ACC_ROWS, OUT), jnp.float32),
        pltpu.SemaphoreType.DMA((NBUF,)),
    ],
    compiler_params=pltpu.CompilerParams(use_tc_tiling_on_sc=False),
  )


NW128 = N * OUT // 128


def _combine_body(p_ref, res_ref, o_ref):
    o_ref[...] = p_ref[0, :NW128] + p_ref[1, :NW128] + res_ref[...]


def _combine(partials, res128):
    p128 = partials.reshape(NC, N_PAD * OUT // 128, 128)
    return pl.pallas_call(
        _combine_body,
        out_shape=jax.ShapeDtypeStruct((NW128, 128), jnp.float32),
    )(p128, res128)


def kernel(x, edge_index, ntype, etype, W_weight, al_weight, ar_weight,
           res_weight, res_bias):
    del ntype, al_weight, ar_weight

    wcat = jnp.transpose(W_weight, (1, 0, 2)).reshape(IN_DIM, NUM_ETYPES * OUT)
    y, res = _typed_mm(x, wcat, res_weight.T, res_bias.reshape(1, OUT))
    ytab = y.reshape(N * NUM_ETYPES, OUT)

    partials = _scatter_kernel()(edge_index, etype, ytab)
    out128 = _combine(partials, res.reshape(NW128, 128))
    return out128.reshape(N, OUT)
